# baseline (speedup 1.0000x reference)
import functools

import jax
import jax.numpy as jnp
from jax import lax
from jax.experimental import pallas as pl
from jax.experimental.pallas import tpu as pltpu
from jax.experimental.pallas import tpu_sc as plsc

N = 10000
D = 256
H = 256
HH = H // 2
E = 160000

NCORE = 2
NSUB = 16
E_B = 128
KB = 80
E_PAD = NSUB * KB * E_B
TOT_B = E_PAD // E_B

Z_ROWS = 632
N_PAD = NSUB * Z_ROWS
CB = 16
NCHUNK = KB // CB

BN = 1000


@functools.cache
def _make_seg_sum():
    mesh = plsc.VectorSubcoreMesh(
        core_axis_name="c", subcore_axis_name="s",
        num_cores=NCORE, num_subcores=NSUB,
    )

    @functools.partial(
        pl.kernel,
        out_type=jax.ShapeDtypeStruct((NCORE, N_PAD, HH), jnp.float32),
        mesh=mesh,
        scratch_types=[
            pltpu.VMEM((CB, E_B), jnp.int32),
            pltpu.VMEM((CB, E_B), jnp.int32),
            pltpu.VMEM((E_B, HH), jnp.float32),
            pltpu.VMEM((E_B, HH), jnp.float32),
            pltpu.VMEM_SHARED((N_PAD, HH), jnp.float32),
            pltpu.SemaphoreType.DMA,
            pltpu.SemaphoreType.DMA,
        ],
    )
    def _seg_sum(src_hbm, dst_hbm, h_hbm, zeros_hbm, out_hbm,
                 src_v, dst_v, rows_a, rows_b, agg_sh, sem_a, sem_b):
        c = lax.axis_index("c")
        s = lax.axis_index("s")
        pltpu.sync_copy(zeros_hbm, agg_sh.at[pl.ds(s * Z_ROWS, Z_ROWS)])
        pltpu.sync_copy(src_hbm.at[pl.ds(s * KB, CB)], src_v)
        pltpu.sync_copy(dst_hbm.at[pl.ds(s * KB, CB)], dst_v)
        plsc.subcore_barrier()

        pltpu.async_copy(h_hbm.at[c].at[src_v.at[0]], rows_a, sem_a)
        pltpu.async_copy(h_hbm.at[c].at[src_v.at[1]], rows_b, sem_b)

        def step(j, rows_cur, sem_cur):
            pltpu.make_async_copy(h_hbm.at[c].at[src_v.at[j]],
                                  rows_cur, sem_cur).wait()

            @pl.when(j + 2 < CB)
            def _():
                pltpu.async_copy(h_hbm.at[c].at[src_v.at[j + 2]],
                                 rows_cur, sem_cur)

            pltpu.sync_copy(rows_cur, agg_sh.at[dst_v.at[j]], add=True)

        def chunk(k, carry):
            def pair(i, carry2):
                step(2 * i, rows_a, sem_a)
                step(2 * i + 1, rows_b, sem_b)
                return carry2

            lax.fori_loop(0, CB // 2, pair, 0)

            @pl.when(k < NCHUNK - 1)
            def _():
                base = s * KB + (k + 1) * CB
                pltpu.sync_copy(src_hbm.at[pl.ds(base, CB)], src_v)
                pltpu.sync_copy(dst_hbm.at[pl.ds(base, CB)], dst_v)
                pltpu.async_copy(h_hbm.at[c].at[src_v.at[0]], rows_a, sem_a)
                pltpu.async_copy(h_hbm.at[c].at[src_v.at[1]], rows_b, sem_b)

            return carry

        lax.fori_loop(0, NCHUNK, chunk, 0)
        plsc.subcore_barrier()
        pltpu.sync_copy(agg_sh.at[pl.ds(s * Z_ROWS, Z_ROWS)],
                        out_hbm.at[c].at[pl.ds(s * Z_ROWS, Z_ROWS)])

    return _seg_sum


def _dense0_body(x_ref, wt_ref, b_ref, out_ref):
    h = jnp.dot(x_ref[...], wt_ref[...], preferred_element_type=jnp.float32)
    h = jnp.maximum(h + b_ref[...], 0.0)
    out_ref[0] = h[:, :HH]
    out_ref[1] = h[:, HH:]


def _pre_body(h_ref, m_ref, out_ref):
    t = jnp.dot(h_ref[0], m_ref[:HH, :], preferred_element_type=jnp.float32)
    t = t + jnp.dot(h_ref[1], m_ref[HH:, :], preferred_element_type=jnp.float32)
    out_ref[0] = t[:, :HH]
    out_ref[1] = t[:, HH:]


def _post_body(final, sc_ref, p_ref, a_ref, m_ref, b_ref, out_ref):
    scale = sc_ref[0, 0]
    t = jnp.dot(a_ref[0], m_ref[:HH, :], preferred_element_type=jnp.float32)
    t = t + jnp.dot(a_ref[1], m_ref[HH:, :], preferred_element_type=jnp.float32)
    t = t + b_ref[...]
    h0 = jnp.maximum(scale * p_ref[0] + t[:, :HH], 0.0)
    h1 = jnp.maximum(scale * p_ref[1] + t[:, HH:], 0.0)
    if final:
        out_ref[:, :HH] = h0
        out_ref[:, HH:] = h1
    else:
        out_ref[0] = h0
        out_ref[1] = h1


_W_SPEC = pl.BlockSpec((D, H), lambda i: (0, 0))
_B_SPEC = pl.BlockSpec((1, H), lambda i: (0, 0))
_H2_SPEC = pl.BlockSpec((NCORE, BN, HH), lambda i: (0, i, 0))

_dense0 = pl.pallas_call(
    _dense0_body,
    grid=(N // BN,),
    in_specs=[pl.BlockSpec((BN, D), lambda i: (i, 0)), _W_SPEC, _B_SPEC],
    out_specs=_H2_SPEC,
    out_shape=jax.ShapeDtypeStruct((NCORE, N, HH), jnp.float32),
)

_AGG_SPEC = pl.BlockSpec((NCORE, BN, HH), lambda i: (0, i, 0))

_pre = pl.pallas_call(
    _pre_body,
    grid=(N // BN,),
    in_specs=[_H2_SPEC, _W_SPEC],
    out_specs=_H2_SPEC,
    out_shape=jax.ShapeDtypeStruct((NCORE, N, HH), jnp.float32),
)

_post_specs = [
    pl.BlockSpec((1, 1), lambda i: (0, 0)),
    _H2_SPEC, _AGG_SPEC, _W_SPEC, _B_SPEC,
]

_post_mid = pl.pallas_call(
    functools.partial(_post_body, False),
    grid=(N // BN,),
    in_specs=_post_specs,
    out_specs=_H2_SPEC,
    out_shape=jax.ShapeDtypeStruct((NCORE, N, HH), jnp.float32),
)

_post_final = pl.pallas_call(
    functools.partial(_post_body, True),
    grid=(N // BN,),
    in_specs=_post_specs,
    out_specs=pl.BlockSpec((BN, H), lambda i: (i, 0)),
    out_shape=jax.ShapeDtypeStruct((N, H), jnp.float32),
)


def _fold_bn(W, b, g, bb, rm, rv):
    s = g * jax.lax.rsqrt(rv + 1e-5)
    return W * s[:, None], b * s + bb - rm * s


def kernel(x, edge_index, fc0_W, fc0_b, gin0_W, gin0_b, gin1_W, gin1_b, W_W, W_b,
           eps0, eps1, bn0_g, bn0_b, bn0_rm, bn0_rv, bn1_g, bn1_b, bn1_rm, bn1_rv,
           bn2_g, bn2_b, bn2_rm, bn2_rv):
    W0, b0 = _fold_bn(fc0_W, fc0_b, bn0_g, bn0_b, bn0_rm, bn0_rv)
    W1, b1 = _fold_bn(W_W, W_b, bn1_g, bn1_b, bn1_rm, bn1_rv)
    W2, b2 = _fold_bn(W_W, W_b, bn2_g, bn2_b, bn2_rm, bn2_rv)
    W0t, b0r = W0.T, b0.reshape(1, H)
    M1 = gin0_W.T @ W1.T
    M2 = gin1_W.T @ W2.T
    bb1 = (gin0_b @ W1.T + b1).reshape(1, H)
    bb2 = (gin1_b @ W2.T + b2).reshape(1, H)
    s0 = (1.0 + eps0).reshape(1, 1)
    s1 = (1.0 + eps1).reshape(1, 1)

    src = jnp.concatenate([edge_index[0], jnp.zeros((E_PAD - E,), jnp.int32)])
    dst = jnp.concatenate([edge_index[1], jnp.full((E_PAD - E,), N, jnp.int32)])
    src2 = src.reshape(TOT_B, E_B)
    dst2 = dst.reshape(TOT_B, E_B)
    zeros = jnp.zeros((Z_ROWS, HH), jnp.float32)

    seg_sum = _make_seg_sum()
    h = _dense0(x, W0t, b0r)
    agg = seg_sum(src2, dst2, h, zeros)
    p = _pre(h, M1)
    h = _post_mid(s0, p, agg, M1, bb1)
    agg = seg_sum(src2, dst2, h, zeros)
    p = _pre(h, M2)
    return _post_final(s1, p, agg, M2, bb2)


# speedup vs baseline: 2.3821x; 2.3821x over previous
import functools

import jax
import jax.numpy as jnp
from jax import lax
from jax.experimental import pallas as pl
from jax.experimental.pallas import tpu as pltpu
from jax.experimental.pallas import tpu_sc as plsc

N = 10000
D = 256
H = 256
HH = H // 2
E = 160000

NCORE = 2
NSUB = 16
E_B = 125
KB = 80
E_PAD = NSUB * KB * E_B
TOT_B = E_PAD // E_B

Z_ROWS = 632
N_PAD = NSUB * Z_ROWS
CB = 16
NCHUNK = KB // CB

BN = 1000


@functools.cache
def _make_seg_sum():
    mesh = plsc.VectorSubcoreMesh(
        core_axis_name="c", subcore_axis_name="s",
        num_cores=NCORE, num_subcores=NSUB,
    )

    @functools.partial(
        pl.kernel,
        out_type=jax.ShapeDtypeStruct((NCORE, N_PAD, HH), jnp.float32),
        mesh=mesh,
        scratch_types=[
            pltpu.VMEM((CB, E_B), jnp.int32),
            pltpu.VMEM((CB, E_B), jnp.int32),
            pltpu.VMEM((E_B, HH), jnp.float32),
            pltpu.VMEM((E_B, HH), jnp.float32),
            pltpu.VMEM_SHARED((N_PAD, HH), jnp.float32),
            pltpu.SemaphoreType.DMA,
            pltpu.SemaphoreType.DMA,
        ],
    )
    def _seg_sum(src_hbm, dst_hbm, h_hbm, zeros_hbm, out_hbm,
                 src_v, dst_v, rows_a, rows_b, agg_sh, sem_a, sem_b):
        c = lax.axis_index("c")
        s = lax.axis_index("s")
        pltpu.sync_copy(zeros_hbm, agg_sh.at[pl.ds(s * Z_ROWS, Z_ROWS)])
        pltpu.sync_copy(src_hbm.at[pl.ds(s * KB, CB)], src_v)
        pltpu.sync_copy(dst_hbm.at[pl.ds(s * KB, CB)], dst_v)
        plsc.subcore_barrier()

        pltpu.async_copy(h_hbm.at[c].at[src_v.at[0]], rows_a, sem_a)
        pltpu.async_copy(h_hbm.at[c].at[src_v.at[1]], rows_b, sem_b)

        def step(j, rows_cur, sem_cur):
            pltpu.make_async_copy(h_hbm.at[c].at[src_v.at[j]],
                                  rows_cur, sem_cur).wait()

            @pl.when(j + 2 < CB)
            def _():
                pltpu.async_copy(h_hbm.at[c].at[src_v.at[j + 2]],
                                 rows_cur, sem_cur)

            pltpu.sync_copy(rows_cur, agg_sh.at[dst_v.at[j]], add=True)

        def chunk(k, carry):
            def pair(i, carry2):
                step(2 * i, rows_a, sem_a)
                step(2 * i + 1, rows_b, sem_b)
                return carry2

            lax.fori_loop(0, CB // 2, pair, 0)

            @pl.when(k < NCHUNK - 1)
            def _():
                base = s * KB + (k + 1) * CB
                pltpu.sync_copy(src_hbm.at[pl.ds(base, CB)], src_v)
                pltpu.sync_copy(dst_hbm.at[pl.ds(base, CB)], dst_v)
                pltpu.async_copy(h_hbm.at[c].at[src_v.at[0]], rows_a, sem_a)
                pltpu.async_copy(h_hbm.at[c].at[src_v.at[1]], rows_b, sem_b)

            return carry

        lax.fori_loop(0, NCHUNK, chunk, 0)
        plsc.subcore_barrier()
        pltpu.sync_copy(agg_sh.at[pl.ds(s * Z_ROWS, Z_ROWS)],
                        out_hbm.at[c].at[pl.ds(s * Z_ROWS, Z_ROWS)])

    return _seg_sum


def _dense0_body(x_ref, wt_ref, b_ref, out_ref):
    h = jnp.dot(x_ref[...], wt_ref[...], preferred_element_type=jnp.float32)
    h = jnp.maximum(h + b_ref[...], 0.0)
    out_ref[0] = h[:, :HH]
    out_ref[1] = h[:, HH:]


def _pre_body(h_ref, m_ref, out_ref):
    t = jnp.dot(h_ref[0], m_ref[:HH, :], preferred_element_type=jnp.float32)
    t = t + jnp.dot(h_ref[1], m_ref[HH:, :], preferred_element_type=jnp.float32)
    out_ref[0] = t[:, :HH]
    out_ref[1] = t[:, HH:]


def _post_body(final, sc_ref, p_ref, a_ref, m_ref, b_ref, out_ref):
    scale = sc_ref[0, 0]
    t = jnp.dot(a_ref[0], m_ref[:HH, :], preferred_element_type=jnp.float32)
    t = t + jnp.dot(a_ref[1], m_ref[HH:, :], preferred_element_type=jnp.float32)
    t = t + b_ref[...]
    h0 = jnp.maximum(scale * p_ref[0] + t[:, :HH], 0.0)
    h1 = jnp.maximum(scale * p_ref[1] + t[:, HH:], 0.0)
    if final:
        out_ref[:, :HH] = h0
        out_ref[:, HH:] = h1
    else:
        out_ref[0] = h0
        out_ref[1] = h1


_W_SPEC = pl.BlockSpec((D, H), lambda i: (0, 0))
_B_SPEC = pl.BlockSpec((1, H), lambda i: (0, 0))
_H2_SPEC = pl.BlockSpec((NCORE, BN, HH), lambda i: (0, i, 0))

_dense0 = pl.pallas_call(
    _dense0_body,
    grid=(N // BN,),
    in_specs=[pl.BlockSpec((BN, D), lambda i: (i, 0)), _W_SPEC, _B_SPEC],
    out_specs=_H2_SPEC,
    out_shape=jax.ShapeDtypeStruct((NCORE, N, HH), jnp.float32),
)

_AGG_SPEC = pl.BlockSpec((NCORE, BN, HH), lambda i: (0, i, 0))

_pre = pl.pallas_call(
    _pre_body,
    grid=(N // BN,),
    in_specs=[_H2_SPEC, _W_SPEC],
    out_specs=_H2_SPEC,
    out_shape=jax.ShapeDtypeStruct((NCORE, N, HH), jnp.float32),
)

_post_specs = [
    pl.BlockSpec((1, 1), lambda i: (0, 0)),
    _H2_SPEC, _AGG_SPEC, _W_SPEC, _B_SPEC,
]

_post_mid = pl.pallas_call(
    functools.partial(_post_body, False),
    grid=(N // BN,),
    in_specs=_post_specs,
    out_specs=_H2_SPEC,
    out_shape=jax.ShapeDtypeStruct((NCORE, N, HH), jnp.float32),
)

_post_final = pl.pallas_call(
    functools.partial(_post_body, True),
    grid=(N // BN,),
    in_specs=_post_specs,
    out_specs=pl.BlockSpec((BN, H), lambda i: (i, 0)),
    out_shape=jax.ShapeDtypeStruct((N, H), jnp.float32),
)


def _fold_bn(W, b, g, bb, rm, rv):
    s = g * jax.lax.rsqrt(rv + 1e-5)
    return W * s[:, None], b * s + bb - rm * s


def kernel(x, edge_index, fc0_W, fc0_b, gin0_W, gin0_b, gin1_W, gin1_b, W_W, W_b,
           eps0, eps1, bn0_g, bn0_b, bn0_rm, bn0_rv, bn1_g, bn1_b, bn1_rm, bn1_rv,
           bn2_g, bn2_b, bn2_rm, bn2_rv):
    W0, b0 = _fold_bn(fc0_W, fc0_b, bn0_g, bn0_b, bn0_rm, bn0_rv)
    W1, b1 = _fold_bn(W_W, W_b, bn1_g, bn1_b, bn1_rm, bn1_rv)
    W2, b2 = _fold_bn(W_W, W_b, bn2_g, bn2_b, bn2_rm, bn2_rv)
    W0t, b0r = W0.T, b0.reshape(1, H)
    M1 = gin0_W.T @ W1.T
    M2 = gin1_W.T @ W2.T
    bb1 = (gin0_b @ W1.T + b1).reshape(1, H)
    bb2 = (gin1_b @ W2.T + b2).reshape(1, H)
    s0 = (1.0 + eps0).reshape(1, 1)
    s1 = (1.0 + eps1).reshape(1, 1)

    src, dst = edge_index[0], edge_index[1]
    if E_PAD > E:
        src = jnp.concatenate([src, jnp.zeros((E_PAD - E,), jnp.int32)])
        dst = jnp.concatenate([dst, jnp.full((E_PAD - E,), N, jnp.int32)])
    src2 = src.reshape(TOT_B, E_B)
    dst2 = dst.reshape(TOT_B, E_B)
    zeros = jnp.zeros((Z_ROWS, HH), jnp.float32)

    seg_sum = _make_seg_sum()
    h = _dense0(x, W0t, b0r)
    agg = seg_sum(src2, dst2, h, zeros)
    p = _pre(h, M1)
    h = _post_mid(s0, p, agg, M1, bb1)
    agg = seg_sum(src2, dst2, h, zeros)
    p = _pre(h, M2)
    return _post_final(s1, p, agg, M2, bb2)


# speedup vs baseline: 2.4177x; 1.0149x over previous
import functools

import jax
import jax.numpy as jnp
from jax import lax
from jax.experimental import pallas as pl
from jax.experimental.pallas import tpu as pltpu
from jax.experimental.pallas import tpu_sc as plsc

N = 10000
D = 256
H = 256
HH = H // 2
E = 160000

NCORE = 2
NSUB = 16
E_B = 125
KB = 80
E_PAD = NSUB * KB * E_B
TOT_B = E_PAD // E_B

Z_ROWS = 632
N_PAD = NSUB * Z_ROWS
CB = 16
NCHUNK = KB // CB

BN = 2000


@functools.cache
def _make_seg_sum():
    mesh = plsc.VectorSubcoreMesh(
        core_axis_name="c", subcore_axis_name="s",
        num_cores=NCORE, num_subcores=NSUB,
    )

    @functools.partial(
        pl.kernel,
        out_type=jax.ShapeDtypeStruct((NCORE, N_PAD, HH), jnp.float32),
        mesh=mesh,
        scratch_types=[
            pltpu.VMEM((CB, E_B), jnp.int32),
            pltpu.VMEM((CB, E_B), jnp.int32),
            pltpu.VMEM((E_B, HH), jnp.float32),
            pltpu.VMEM((E_B, HH), jnp.float32),
            pltpu.VMEM_SHARED((N_PAD, HH), jnp.float32),
            pltpu.SemaphoreType.DMA,
            pltpu.SemaphoreType.DMA,
        ],
    )
    def _seg_sum(src_hbm, dst_hbm, h_hbm, zeros_hbm, out_hbm,
                 src_v, dst_v, rows_a, rows_b, agg_sh, sem_a, sem_b):
        c = lax.axis_index("c")
        s = lax.axis_index("s")
        pltpu.sync_copy(zeros_hbm, agg_sh.at[pl.ds(s * Z_ROWS, Z_ROWS)])
        pltpu.sync_copy(src_hbm.at[pl.ds(s * KB, CB)], src_v)
        pltpu.sync_copy(dst_hbm.at[pl.ds(s * KB, CB)], dst_v)
        plsc.subcore_barrier()

        pltpu.async_copy(h_hbm.at[c].at[src_v.at[0]], rows_a, sem_a)
        pltpu.async_copy(h_hbm.at[c].at[src_v.at[1]], rows_b, sem_b)

        def step(j, rows_cur, sem_cur):
            pltpu.make_async_copy(h_hbm.at[c].at[src_v.at[j]],
                                  rows_cur, sem_cur).wait()

            @pl.when(j + 2 < CB)
            def _():
                pltpu.async_copy(h_hbm.at[c].at[src_v.at[j + 2]],
                                 rows_cur, sem_cur)

            pltpu.sync_copy(rows_cur, agg_sh.at[dst_v.at[j]], add=True)

        def chunk(k, carry):
            def pair(i, carry2):
                step(2 * i, rows_a, sem_a)
                step(2 * i + 1, rows_b, sem_b)
                return carry2

            lax.fori_loop(0, CB // 2, pair, 0)

            @pl.when(k < NCHUNK - 1)
            def _():
                base = s * KB + (k + 1) * CB
                pltpu.sync_copy(src_hbm.at[pl.ds(base, CB)], src_v)
                pltpu.sync_copy(dst_hbm.at[pl.ds(base, CB)], dst_v)
                pltpu.async_copy(h_hbm.at[c].at[src_v.at[0]], rows_a, sem_a)
                pltpu.async_copy(h_hbm.at[c].at[src_v.at[1]], rows_b, sem_b)

            return carry

        lax.fori_loop(0, NCHUNK, chunk, 0)
        plsc.subcore_barrier()
        pltpu.sync_copy(agg_sh.at[pl.ds(s * Z_ROWS, Z_ROWS)],
                        out_hbm.at[c].at[pl.ds(s * Z_ROWS, Z_ROWS)])

    return _seg_sum


def _dense0_body(x_ref, wt_ref, b_ref, out_ref):
    h = jnp.dot(x_ref[...], wt_ref[...], preferred_element_type=jnp.float32)
    h = jnp.maximum(h + b_ref[...], 0.0)
    out_ref[0] = h[:, :HH]
    out_ref[1] = h[:, HH:]


def _pre_body(h_ref, m_ref, out_ref):
    t = jnp.dot(h_ref[0], m_ref[:HH, :], preferred_element_type=jnp.float32)
    t = t + jnp.dot(h_ref[1], m_ref[HH:, :], preferred_element_type=jnp.float32)
    out_ref[0] = t[:, :HH]
    out_ref[1] = t[:, HH:]


def _post_body(final, sc_ref, p_ref, a_ref, m_ref, b_ref, out_ref):
    scale = sc_ref[0, 0]
    t = jnp.dot(a_ref[0], m_ref[:HH, :], preferred_element_type=jnp.float32)
    t = t + jnp.dot(a_ref[1], m_ref[HH:, :], preferred_element_type=jnp.float32)
    t = t + b_ref[...]
    h0 = jnp.maximum(scale * p_ref[0] + t[:, :HH], 0.0)
    h1 = jnp.maximum(scale * p_ref[1] + t[:, HH:], 0.0)
    if final:
        out_ref[:, :HH] = h0
        out_ref[:, HH:] = h1
    else:
        out_ref[0] = h0
        out_ref[1] = h1


_W_SPEC = pl.BlockSpec((D, H), lambda i: (0, 0))
_B_SPEC = pl.BlockSpec((1, H), lambda i: (0, 0))
_H2_SPEC = pl.BlockSpec((NCORE, BN, HH), lambda i: (0, i, 0))

_dense0 = pl.pallas_call(
    _dense0_body,
    grid=(N // BN,),
    in_specs=[pl.BlockSpec((BN, D), lambda i: (i, 0)), _W_SPEC, _B_SPEC],
    out_specs=_H2_SPEC,
    out_shape=jax.ShapeDtypeStruct((NCORE, N, HH), jnp.float32),
)

_AGG_SPEC = pl.BlockSpec((NCORE, BN, HH), lambda i: (0, i, 0))

_pre = pl.pallas_call(
    _pre_body,
    grid=(N // BN,),
    in_specs=[_H2_SPEC, _W_SPEC],
    out_specs=_H2_SPEC,
    out_shape=jax.ShapeDtypeStruct((NCORE, N, HH), jnp.float32),
)

_post_specs = [
    pl.BlockSpec((1, 1), lambda i: (0, 0)),
    _H2_SPEC, _AGG_SPEC, _W_SPEC, _B_SPEC,
]

_post_mid = pl.pallas_call(
    functools.partial(_post_body, False),
    grid=(N // BN,),
    in_specs=_post_specs,
    out_specs=_H2_SPEC,
    out_shape=jax.ShapeDtypeStruct((NCORE, N, HH), jnp.float32),
)

_post_final = pl.pallas_call(
    functools.partial(_post_body, True),
    grid=(N // BN,),
    in_specs=_post_specs,
    out_specs=pl.BlockSpec((BN, H), lambda i: (i, 0)),
    out_shape=jax.ShapeDtypeStruct((N, H), jnp.float32),
)


def _fold_bn(W, b, g, bb, rm, rv):
    s = g * jax.lax.rsqrt(rv + 1e-5)
    return W * s[:, None], b * s + bb - rm * s


def kernel(x, edge_index, fc0_W, fc0_b, gin0_W, gin0_b, gin1_W, gin1_b, W_W, W_b,
           eps0, eps1, bn0_g, bn0_b, bn0_rm, bn0_rv, bn1_g, bn1_b, bn1_rm, bn1_rv,
           bn2_g, bn2_b, bn2_rm, bn2_rv):
    W0, b0 = _fold_bn(fc0_W, fc0_b, bn0_g, bn0_b, bn0_rm, bn0_rv)
    W1, b1 = _fold_bn(W_W, W_b, bn1_g, bn1_b, bn1_rm, bn1_rv)
    W2, b2 = _fold_bn(W_W, W_b, bn2_g, bn2_b, bn2_rm, bn2_rv)
    W0t, b0r = W0.T, b0.reshape(1, H)
    M1 = gin0_W.T @ W1.T
    M2 = gin1_W.T @ W2.T
    bb1 = (gin0_b @ W1.T + b1).reshape(1, H)
    bb2 = (gin1_b @ W2.T + b2).reshape(1, H)
    s0 = (1.0 + eps0).reshape(1, 1)
    s1 = (1.0 + eps1).reshape(1, 1)

    src, dst = edge_index[0], edge_index[1]
    if E_PAD > E:
        src = jnp.concatenate([src, jnp.zeros((E_PAD - E,), jnp.int32)])
        dst = jnp.concatenate([dst, jnp.full((E_PAD - E,), N, jnp.int32)])
    src2 = src.reshape(TOT_B, E_B)
    dst2 = dst.reshape(TOT_B, E_B)
    zeros = jnp.zeros((Z_ROWS, HH), jnp.float32)

    seg_sum = _make_seg_sum()
    h = _dense0(x, W0t, b0r)
    agg = seg_sum(src2, dst2, h, zeros)
    p = _pre(h, M1)
    h = _post_mid(s0, p, agg, M1, bb1)
    agg = seg_sum(src2, dst2, h, zeros)
    p = _pre(h, M2)
    return _post_final(s1, p, agg, M2, bb2)


# speedup vs baseline: 2.6056x; 1.0777x over previous
import functools

import jax
import jax.numpy as jnp
from jax import lax
from jax.experimental import pallas as pl
from jax.experimental.pallas import tpu as pltpu
from jax.experimental.pallas import tpu_sc as plsc

N = 10000
D = 256
H = 256
HH = H // 2
E = 160000

NCORE = 2
NSUB = 16
E_B = 125
KB = 80
E_PAD = NSUB * KB * E_B
TOT_B = E_PAD // E_B

Z_ROWS = 632
N_PAD = NSUB * Z_ROWS
CB = 8
NCHUNK = KB // CB

BN = 2000


@functools.cache
def _make_seg_sum():
    mesh = plsc.VectorSubcoreMesh(
        core_axis_name="c", subcore_axis_name="s",
        num_cores=NCORE, num_subcores=NSUB,
    )

    @functools.partial(
        pl.kernel,
        out_type=jax.ShapeDtypeStruct((NCORE, N_PAD, HH), jnp.float32),
        mesh=mesh,
        scratch_types=[
            pltpu.VMEM((CB, E_B), jnp.int32),
            pltpu.VMEM((CB, E_B), jnp.int32),
            pltpu.VMEM((CB, E_B), jnp.int32),
            pltpu.VMEM((CB, E_B), jnp.int32),
            pltpu.VMEM((E_B, HH), jnp.float32),
            pltpu.VMEM((E_B, HH), jnp.float32),
            pltpu.VMEM_SHARED((N_PAD, HH), jnp.float32),
            pltpu.SemaphoreType.DMA,
            pltpu.SemaphoreType.DMA,
            pltpu.SemaphoreType.DMA,
        ],
    )
    def _seg_sum(src_hbm, dst_hbm, h_hbm, zeros_hbm, out_hbm,
                 src0, src1, dst0, dst1, rows_a, rows_b, agg_sh,
                 sem_a, sem_b, sem_i):
        c = lax.axis_index("c")
        s = lax.axis_index("s")
        pltpu.sync_copy(zeros_hbm, agg_sh.at[pl.ds(s * Z_ROWS, Z_ROWS)])
        pltpu.sync_copy(src_hbm.at[pl.ds(s * KB, CB)], src0)
        pltpu.sync_copy(dst_hbm.at[pl.ds(s * KB, CB)], dst0)
        plsc.subcore_barrier()

        pltpu.async_copy(h_hbm.at[c].at[src0.at[0]], rows_a, sem_a)
        pltpu.async_copy(h_hbm.at[c].at[src0.at[1]], rows_b, sem_b)

        rows = (rows_a, rows_b)
        sems = (sem_a, sem_b)

        def do_chunk(k, src_c, dst_c, src_n, dst_n, prefetch):
            @pl.when(prefetch)
            def _():
                base = (k + 1) * CB + s * KB
                pltpu.async_copy(src_hbm.at[pl.ds(base, CB)], src_n, sem_i)
                pltpu.async_copy(dst_hbm.at[pl.ds(base, CB)], dst_n, sem_i)

            for j in range(CB):
                rj, sj = rows[j % 2], sems[j % 2]
                pltpu.make_async_copy(h_hbm.at[c].at[src_c.at[j]],
                                      rj, sj).wait()
                if j + 2 < CB:
                    pltpu.async_copy(h_hbm.at[c].at[src_c.at[j + 2]], rj, sj)
                else:
                    @pl.when(prefetch)
                    def _():
                        if j == CB - 2:
                            pltpu.make_async_copy(
                                src_hbm.at[pl.ds(s * KB, CB)], src_n,
                                sem_i).wait()
                            pltpu.make_async_copy(
                                dst_hbm.at[pl.ds(s * KB, CB)], dst_n,
                                sem_i).wait()
                        pltpu.async_copy(h_hbm.at[c].at[src_n.at[j - (CB - 2)]],
                                         rj, sj)

                pltpu.sync_copy(rj, agg_sh.at[dst_c.at[j]], add=True)

        def superchunk(m, carry):
            do_chunk(2 * m, src0, dst0, src1, dst1, m >= 0)
            do_chunk(2 * m + 1, src1, dst1, src0, dst0, m < NCHUNK // 2 - 1)
            return carry

        lax.fori_loop(0, NCHUNK // 2, superchunk, 0)
        plsc.subcore_barrier()
        pltpu.sync_copy(agg_sh.at[pl.ds(s * Z_ROWS, Z_ROWS)],
                        out_hbm.at[c].at[pl.ds(s * Z_ROWS, Z_ROWS)])

    return _seg_sum


def _dense0_body(x_ref, wt_ref, b_ref, out_ref):
    h = jnp.dot(x_ref[...], wt_ref[...], preferred_element_type=jnp.float32)
    h = jnp.maximum(h + b_ref[...], 0.0)
    out_ref[0] = h[:, :HH]
    out_ref[1] = h[:, HH:]


def _pre_body(h_ref, m_ref, out_ref):
    t = jnp.dot(h_ref[0], m_ref[:HH, :], preferred_element_type=jnp.float32)
    t = t + jnp.dot(h_ref[1], m_ref[HH:, :], preferred_element_type=jnp.float32)
    out_ref[0] = t[:, :HH]
    out_ref[1] = t[:, HH:]


def _post_body(final, sc_ref, p_ref, a_ref, m_ref, b_ref, out_ref):
    scale = sc_ref[0, 0]
    t = jnp.dot(a_ref[0], m_ref[:HH, :], preferred_element_type=jnp.float32)
    t = t + jnp.dot(a_ref[1], m_ref[HH:, :], preferred_element_type=jnp.float32)
    t = t + b_ref[...]
    h0 = jnp.maximum(scale * p_ref[0] + t[:, :HH], 0.0)
    h1 = jnp.maximum(scale * p_ref[1] + t[:, HH:], 0.0)
    if final:
        out_ref[:, :HH] = h0
        out_ref[:, HH:] = h1
    else:
        out_ref[0] = h0
        out_ref[1] = h1


_W_SPEC = pl.BlockSpec((D, H), lambda i: (0, 0))
_B_SPEC = pl.BlockSpec((1, H), lambda i: (0, 0))
_H2_SPEC = pl.BlockSpec((NCORE, BN, HH), lambda i: (0, i, 0))

_dense0 = pl.pallas_call(
    _dense0_body,
    grid=(N // BN,),
    in_specs=[pl.BlockSpec((BN, D), lambda i: (i, 0)), _W_SPEC, _B_SPEC],
    out_specs=_H2_SPEC,
    out_shape=jax.ShapeDtypeStruct((NCORE, N, HH), jnp.float32),
)

_AGG_SPEC = pl.BlockSpec((NCORE, BN, HH), lambda i: (0, i, 0))

_pre = pl.pallas_call(
    _pre_body,
    grid=(N // BN,),
    in_specs=[_H2_SPEC, _W_SPEC],
    out_specs=_H2_SPEC,
    out_shape=jax.ShapeDtypeStruct((NCORE, N, HH), jnp.float32),
)

_post_specs = [
    pl.BlockSpec((1, 1), lambda i: (0, 0)),
    _H2_SPEC, _AGG_SPEC, _W_SPEC, _B_SPEC,
]

_post_mid = pl.pallas_call(
    functools.partial(_post_body, False),
    grid=(N // BN,),
    in_specs=_post_specs,
    out_specs=_H2_SPEC,
    out_shape=jax.ShapeDtypeStruct((NCORE, N, HH), jnp.float32),
)

_post_final = pl.pallas_call(
    functools.partial(_post_body, True),
    grid=(N // BN,),
    in_specs=_post_specs,
    out_specs=pl.BlockSpec((BN, H), lambda i: (i, 0)),
    out_shape=jax.ShapeDtypeStruct((N, H), jnp.float32),
)


def _fold_bn(W, b, g, bb, rm, rv):
    s = g * jax.lax.rsqrt(rv + 1e-5)
    return W * s[:, None], b * s + bb - rm * s


def kernel(x, edge_index, fc0_W, fc0_b, gin0_W, gin0_b, gin1_W, gin1_b, W_W, W_b,
           eps0, eps1, bn0_g, bn0_b, bn0_rm, bn0_rv, bn1_g, bn1_b, bn1_rm, bn1_rv,
           bn2_g, bn2_b, bn2_rm, bn2_rv):
    W0, b0 = _fold_bn(fc0_W, fc0_b, bn0_g, bn0_b, bn0_rm, bn0_rv)
    W1, b1 = _fold_bn(W_W, W_b, bn1_g, bn1_b, bn1_rm, bn1_rv)
    W2, b2 = _fold_bn(W_W, W_b, bn2_g, bn2_b, bn2_rm, bn2_rv)
    W0t, b0r = W0.T, b0.reshape(1, H)
    M1 = gin0_W.T @ W1.T
    M2 = gin1_W.T @ W2.T
    bb1 = (gin0_b @ W1.T + b1).reshape(1, H)
    bb2 = (gin1_b @ W2.T + b2).reshape(1, H)
    s0 = (1.0 + eps0).reshape(1, 1)
    s1 = (1.0 + eps1).reshape(1, 1)

    src, dst = edge_index[0], edge_index[1]
    if E_PAD > E:
        src = jnp.concatenate([src, jnp.zeros((E_PAD - E,), jnp.int32)])
        dst = jnp.concatenate([dst, jnp.full((E_PAD - E,), N, jnp.int32)])
    src2 = src.reshape(TOT_B, E_B)
    dst2 = dst.reshape(TOT_B, E_B)
    zeros = jnp.zeros((Z_ROWS, HH), jnp.float32)

    seg_sum = _make_seg_sum()
    h = _dense0(x, W0t, b0r)
    agg = seg_sum(src2, dst2, h, zeros)
    p = _pre(h, M1)
    h = _post_mid(s0, p, agg, M1, bb1)
    agg = seg_sum(src2, dst2, h, zeros)
    p = _pre(h, M2)
    return _post_final(s1, p, agg, M2, bb2)
